# Initial kernel scaffold; baseline (speedup 1.0000x reference)
import jax
import jax.numpy as jnp
from jax.experimental import pallas as pl


def kernel(embedding, r_embedding):
    raise NotImplementedError("write your pallas kernel here")


# speedup vs baseline: 2.6510x; 2.6510x over previous
import jax
import jax.numpy as jnp
from jax.experimental import pallas as pl

N = 8192
BR = 256
GRID = N // BR
BIG = 3.0e38


def _body(emb_ref, r_ref, val_ref, idx_ref):
    i = pl.program_id(0)
    scaled = r_ref[...] * emb_ref[...]
    bmax = jnp.max(scaled, axis=0, keepdims=True)
    rows = jax.lax.broadcasted_iota(jnp.int32, (BR, N), 0) + i * BR
    cand = jnp.where(scaled == bmax, rows, jnp.int32(2**30))
    bidx = jnp.min(cand, axis=0, keepdims=True).astype(jnp.float32)

    @pl.when(i == 0)
    def _init():
        val_ref[...] = bmax
        idx_ref[...] = bidx

    @pl.when(i > 0)
    def _acc():
        upd = bmax > val_ref[...]
        idx_ref[...] = jnp.where(upd, bidx, idx_ref[...])
        val_ref[...] = jnp.maximum(val_ref[...], bmax)

    @pl.when(i == GRID - 1)
    def _final():
        idx_ref[...] = jnp.where(val_ref[...] > 0.0, idx_ref[...], 0.0)


def kernel(embedding, r_embedding):
    emb_t = embedding.reshape(N, 1)
    val, idx = pl.pallas_call(
        _body,
        grid=(GRID,),
        in_specs=[
            pl.BlockSpec((BR, 1), lambda i: (i, 0)),
            pl.BlockSpec((BR, N), lambda i: (i, 0)),
        ],
        out_specs=[
            pl.BlockSpec((1, N), lambda i: (0, 0)),
            pl.BlockSpec((1, N), lambda i: (0, 0)),
        ],
        out_shape=[
            jax.ShapeDtypeStruct((1, N), jnp.float32),
            jax.ShapeDtypeStruct((1, N), jnp.float32),
        ],
    )(emb_t, r_embedding)
    return val, idx.reshape(N)
